# Initial kernel scaffold; baseline (speedup 1.0000x reference)
import jax
import jax.numpy as jnp
from jax.experimental import pallas as pl


def kernel(x, edge_index, W1, att_src1, att_dst1, b1, W2, att_src2, att_dst2, b2):
    raise NotImplementedError("write your pallas kernel here")


# speedup vs baseline: 36.7512x; 36.7512x over previous
import functools

import jax
import jax.numpy as jnp
from jax import lax
from jax.experimental import pallas as pl
from jax.experimental.pallas import tpu as pltpu
from jax.experimental.pallas import tpu_sc as plsc

N = 10000
E = 320000
D = 128
HID = 8
HEADS = 8
C = 16

NR = 10240
EPAD = 327680
SUBLEN = 128
SUB = 4
K = SUB * SUBLEN
IDXROWS = EPAD // SUBLEN
ROWS_PT = NR // 16

W1A = 48
WAD = 16
W2A = 32
EPS = 1e-16

_BLK = 512
_GRID = NR // _BLK


def _tc1_body(x_ref, w1_ref, a1_ref, h1s_ref, ads_ref):
    h = jnp.dot(x_ref[...], w1_ref[...], preferred_element_type=jnp.float32)
    s = jnp.dot(h, a1_ref[...], preferred_element_type=jnp.float32)
    z12 = jnp.zeros((h.shape[0], 12), jnp.float32)
    h1s_ref[0] = jnp.concatenate([h[:, :32], s[:, 0:4], z12], axis=1)
    h1s_ref[1] = jnp.concatenate([h[:, 32:64], s[:, 4:8], z12], axis=1)
    ads_ref[0] = jnp.concatenate([s[:, 8:12], z12], axis=1)
    ads_ref[1] = jnp.concatenate([s[:, 12:16], z12], axis=1)


def _tc2_body(p_ref, b1_ref, w2_ref, at2_ref, h2a_ref, ad2_ref):
    p0 = p_ref[0]
    p1 = p_ref[1]
    num = jnp.concatenate([p0[:, :32], p1[:, :32]], axis=1)
    den = jnp.concatenate([p0[:, 32:36], p1[:, 32:36]], axis=1)
    hh = lax.broadcasted_iota(jnp.int32, (8, 64), 0)
    cc = lax.broadcasted_iota(jnp.int32, (8, 64), 1) // 8
    rep = (hh == cc).astype(jnp.float32)
    den64 = jnp.dot(den, rep, preferred_element_type=jnp.float32)
    out1 = num / (den64 + EPS) + b1_ref[...]
    x2 = jnp.where(out1 > 0, out1, jnp.exp(jnp.minimum(out1, 0.0)) - 1.0)
    h2 = jnp.dot(x2, w2_ref[...], preferred_element_type=jnp.float32)
    a2 = jnp.dot(h2, at2_ref[...], preferred_element_type=jnp.float32)
    z14 = jnp.zeros((h2.shape[0], 14), jnp.float32)
    z15 = jnp.zeros((h2.shape[0], 15), jnp.float32)
    h2a_ref[...] = jnp.concatenate([h2, a2, z14], axis=1)
    ad2_ref[...] = jnp.concatenate([a2[:, 1:2], z15], axis=1)


def _tc3_body(q_ref, b2_ref, o_ref):
    q0 = q_ref[0]
    q1 = q_ref[1]
    num = q0[:, :16] + q1[:, :16]
    den = q0[:, 16:17] + q1[:, 16:17]
    o = num / (den + EPS) + b2_ref[...]
    m = jnp.max(o, axis=1, keepdims=True)
    z = o - m
    lse = jnp.log(jnp.sum(jnp.exp(z), axis=1, keepdims=True))
    o_ref[...] = z - lse


_MESH = plsc.VectorSubcoreMesh(core_axis_name="c", subcore_axis_name="s")

_GDN = lax.GatherDimensionNumbers(
    offset_dims=(), collapsed_slice_dims=(0,), start_index_map=(0,))


def _vgat(v, idx):
    return lax.gather(v, idx[:, None], _GDN, (1,),
                      mode=lax.GatherScatterMode.PROMISE_IN_BOUNDS)


def _zero_acc(msg_v, acc, s, width):
    z = jnp.zeros((16,), jnp.float32)

    def zb(i, _):
        for q in range(width // 16):
            msg_v[i, pl.ds(q * 16, 16)] = z
        return 0

    lax.fori_loop(0, K, zb, 0)
    r0 = s * ROWS_PT
    pltpu.sync_copy(msg_v, acc.at[pl.ds(r0, K)])
    pltpu.sync_copy(msg_v.at[pl.ds(0, ROWS_PT - K)],
                    acc.at[pl.ds(r0 + K, ROWS_PT - K)])


def _acc_out(acc, out_hbm, c, s):
    r0 = s * ROWS_PT
    pltpu.sync_copy(acc.at[pl.ds(r0, K)], out_hbm.at[c].at[pl.ds(r0, K)])
    pltpu.sync_copy(acc.at[pl.ds(r0 + K, ROWS_PT - K)],
                    out_hbm.at[c].at[pl.ds(r0 + K, ROWS_PT - K)])


@functools.partial(
    pl.kernel,
    mesh=_MESH,
    compiler_params=pltpu.CompilerParams(use_tc_tiling_on_sc=False),
    out_type=jax.ShapeDtypeStruct((2, NR, W1A), jnp.float32),
    scratch_types=[
        pltpu.VMEM((SUB, SUBLEN), jnp.int32),
        pltpu.VMEM((SUB, SUBLEN), jnp.int32),
        pltpu.VMEM((K, W1A), jnp.float32),
        pltpu.VMEM((K, WAD), jnp.float32),
        pltpu.VMEM((K, W1A), jnp.float32),
        pltpu.VMEM_SHARED((NR, W1A), jnp.float32),
        pltpu.SemaphoreType.DMA,
    ],
)
def _sc1(src_hbm, dst_hbm, h1s_hbm, ads_hbm, out_hbm,
         idx_s, idx_d, h1a_v, ad_v, msg_v, acc, sem):
    c = lax.axis_index("c")
    s = lax.axis_index("s")
    _zero_acc(msg_v, acc, s, W1A)
    plsc.subcore_barrier()

    h1a_src = h1s_hbm.at[c]
    ad_src = ads_hbm.at[c]
    lane = lax.iota(jnp.int32, 16)
    hsel = lax.shift_right_logical(lane, 3)
    ebase = s * (EPAD // 16 // SUBLEN)

    def round_body(r, _):
        row = ebase + r * SUB
        pltpu.sync_copy(src_hbm.at[pl.ds(row, SUB)], idx_s)
        pltpu.sync_copy(dst_hbm.at[pl.ds(row, SUB)], idx_d)
        for j in range(SUB):
            pltpu.async_copy(h1a_src.at[idx_s.at[j]],
                             h1a_v.at[pl.ds(j * SUBLEN, SUBLEN)], sem).wait()
            pltpu.async_copy(ad_src.at[idx_d.at[j]],
                             ad_v.at[pl.ds(j * SUBLEN, SUBLEN)], sem).wait()

        def ebody(i, _):
            sa = h1a_v[i, pl.ds(32, 16)]
            da = ad_v[i, pl.ds(0, 16)]
            e = jnp.minimum(sa + da, 60.0)
            e = jnp.where(e >= 0.0, e, 0.2 * e)
            ex = jnp.exp(e)
            m0 = h1a_v[i, pl.ds(0, 16)] * _vgat(ex, hsel)
            m1 = h1a_v[i, pl.ds(16, 16)] * _vgat(ex, hsel + 2)
            msg_v[i, pl.ds(0, 16)] = m0
            msg_v[i, pl.ds(16, 16)] = m1
            msg_v[i, pl.ds(32, 16)] = ex
            return 0

        lax.fori_loop(0, K, ebody, 0)
        for j in range(SUB):
            pltpu.sync_copy(msg_v.at[pl.ds(j * SUBLEN, SUBLEN)],
                            acc.at[idx_d.at[j]], add=True)
        return 0

    lax.fori_loop(0, EPAD // 16 // K, round_body, 0)
    plsc.subcore_barrier()
    _acc_out(acc, out_hbm, c, s)


@functools.partial(
    pl.kernel,
    mesh=_MESH,
    compiler_params=pltpu.CompilerParams(use_tc_tiling_on_sc=False),
    out_type=jax.ShapeDtypeStruct((2, NR, W2A), jnp.float32),
    scratch_types=[
        pltpu.VMEM((SUB, SUBLEN), jnp.int32),
        pltpu.VMEM((SUB, SUBLEN), jnp.int32),
        pltpu.VMEM((K, W2A), jnp.float32),
        pltpu.VMEM((K, WAD), jnp.float32),
        pltpu.VMEM((K, W2A), jnp.float32),
        pltpu.VMEM_SHARED((NR, W2A), jnp.float32),
        pltpu.SemaphoreType.DMA,
    ],
)
def _sc2(src_hbm, dst_hbm, h2a_hbm, ad_hbm, out_hbm,
         idx_s, idx_d, h2a_v, ad_v, msg_v, acc, sem):
    c = lax.axis_index("c")
    s = lax.axis_index("s")
    wid = c * 16 + s
    _zero_acc(msg_v, acc, s, W2A)
    plsc.subcore_barrier()

    zsel = jnp.zeros((16,), jnp.int32)
    ebase = wid * (EPAD // 32 // SUBLEN)

    def round_body(r, _):
        row = ebase + r * SUB
        pltpu.sync_copy(src_hbm.at[pl.ds(row, SUB)], idx_s)
        pltpu.sync_copy(dst_hbm.at[pl.ds(row, SUB)], idx_d)
        for j in range(SUB):
            pltpu.async_copy(h2a_hbm.at[idx_s.at[j]],
                             h2a_v.at[pl.ds(j * SUBLEN, SUBLEN)], sem).wait()
            pltpu.async_copy(ad_hbm.at[idx_d.at[j]],
                             ad_v.at[pl.ds(j * SUBLEN, SUBLEN)], sem).wait()

        def ebody(i, _):
            sa = h2a_v[i, pl.ds(16, 16)]
            da = ad_v[i, pl.ds(0, 16)]
            e = jnp.minimum(sa + da, 60.0)
            e = jnp.where(e >= 0.0, e, 0.2 * e)
            ex = jnp.exp(e)
            hv = h2a_v[i, pl.ds(0, 16)]
            msg_v[i, pl.ds(0, 16)] = hv * _vgat(ex, zsel)
            msg_v[i, pl.ds(16, 16)] = ex
            return 0

        lax.fori_loop(0, K, ebody, 0)
        for j in range(SUB):
            pltpu.sync_copy(msg_v.at[pl.ds(j * SUBLEN, SUBLEN)],
                            acc.at[idx_d.at[j]], add=True)
        return 0

    lax.fori_loop(0, EPAD // 32 // K, round_body, 0)
    plsc.subcore_barrier()
    _acc_out(acc, out_hbm, c, s)


def kernel(x, edge_index, W1, att_src1, att_dst1, b1, W2, att_src2, att_dst2, b2):
    f32 = jnp.float32
    xp = jnp.pad(x.astype(f32), ((0, NR - N), (0, 0)))
    pad = jnp.full((EPAD - E,), N, jnp.int32)
    src2d = jnp.concatenate([edge_index[0], pad]).reshape(IDXROWS, SUBLEN)
    dst2d = jnp.concatenate([edge_index[1], pad]).reshape(IDXROWS, SUBLEN)

    r64 = jnp.arange(64)
    a1 = jnp.zeros((64, 16), f32)
    a1 = a1.at[r64, r64 // 8].set(att_src1.reshape(64))
    a1 = a1.at[r64, 8 + r64 // 8].set(att_dst1.reshape(64))
    at2 = jnp.concatenate([att_src2, att_dst2], axis=0).T

    h1s, ads = pl.pallas_call(
        _tc1_body,
        grid=(_GRID,),
        in_specs=[
            pl.BlockSpec((_BLK, D), lambda i: (i, 0)),
            pl.BlockSpec((D, 64), lambda i: (0, 0)),
            pl.BlockSpec((64, 16), lambda i: (0, 0)),
        ],
        out_specs=[
            pl.BlockSpec((2, _BLK, W1A), lambda i: (0, i, 0)),
            pl.BlockSpec((2, _BLK, WAD), lambda i: (0, i, 0)),
        ],
        out_shape=[
            jax.ShapeDtypeStruct((2, NR, W1A), f32),
            jax.ShapeDtypeStruct((2, NR, WAD), f32),
        ],
    )(xp, W1, a1)

    p = _sc1(src2d, dst2d, h1s, ads)

    h2a, ad2 = pl.pallas_call(
        _tc2_body,
        grid=(_GRID,),
        in_specs=[
            pl.BlockSpec((2, _BLK, W1A), lambda i: (0, i, 0)),
            pl.BlockSpec((1, 64), lambda i: (0, 0)),
            pl.BlockSpec((64, C), lambda i: (0, 0)),
            pl.BlockSpec((C, 2), lambda i: (0, 0)),
        ],
        out_specs=[
            pl.BlockSpec((_BLK, W2A), lambda i: (i, 0)),
            pl.BlockSpec((_BLK, WAD), lambda i: (i, 0)),
        ],
        out_shape=[
            jax.ShapeDtypeStruct((NR, W2A), f32),
            jax.ShapeDtypeStruct((NR, WAD), f32),
        ],
    )(p, b1.reshape(1, 64), W2, at2)

    q = _sc2(src2d, dst2d, h2a, ad2)

    out = pl.pallas_call(
        _tc3_body,
        grid=(_GRID,),
        in_specs=[
            pl.BlockSpec((2, _BLK, W2A), lambda i: (0, i, 0)),
            pl.BlockSpec((1, C), lambda i: (0, 0)),
        ],
        out_specs=pl.BlockSpec((_BLK, C), lambda i: (i, 0)),
        out_shape=jax.ShapeDtypeStruct((NR, C), f32),
    )(q, b2.reshape(1, C))

    return out[:N]


# speedup vs baseline: 66.7386x; 1.8160x over previous
import functools

import jax
import jax.numpy as jnp
from jax import lax
from jax.experimental import pallas as pl
from jax.experimental.pallas import tpu as pltpu
from jax.experimental.pallas import tpu_sc as plsc

N = 10000
E = 320000
D = 128
HID = 8
HEADS = 8
C = 16

NR = 10240
EPAD = 327680
SUBLEN = 128
SUB = 4
K = SUB * SUBLEN
IDXROWS = EPAD // SUBLEN
ROWS_PT = NR // 16

W1A = 48
WAD = 16
W2A = 32
EPS = 1e-16

_BLK = 512
_GRID = NR // _BLK


def _tc1_body(x_ref, w1_ref, a1_ref, h1s_ref, ads_ref):
    h = jnp.dot(x_ref[...], w1_ref[...], preferred_element_type=jnp.float32)
    s = jnp.dot(h, a1_ref[...], preferred_element_type=jnp.float32)
    z12 = jnp.zeros((h.shape[0], 12), jnp.float32)
    h1s_ref[0] = jnp.concatenate([h[:, :32], s[:, 0:4], z12], axis=1)
    h1s_ref[1] = jnp.concatenate([h[:, 32:64], s[:, 4:8], z12], axis=1)
    ads_ref[0] = jnp.concatenate([s[:, 8:12], z12], axis=1)
    ads_ref[1] = jnp.concatenate([s[:, 12:16], z12], axis=1)


def _tc2_body(p_ref, b1_ref, w2_ref, at2_ref, h2a_ref, ad2_ref):
    p0 = p_ref[0]
    p1 = p_ref[1]
    num = jnp.concatenate([p0[:, :32], p1[:, :32]], axis=1)
    den = jnp.concatenate([p0[:, 32:36], p1[:, 32:36]], axis=1)
    hh = lax.broadcasted_iota(jnp.int32, (8, 64), 0)
    cc = lax.broadcasted_iota(jnp.int32, (8, 64), 1) // 8
    rep = (hh == cc).astype(jnp.float32)
    den64 = jnp.dot(den, rep, preferred_element_type=jnp.float32)
    out1 = num / (den64 + EPS) + b1_ref[...]
    x2 = jnp.where(out1 > 0, out1, jnp.exp(jnp.minimum(out1, 0.0)) - 1.0)
    h2 = jnp.dot(x2, w2_ref[...], preferred_element_type=jnp.float32)
    a2 = jnp.dot(h2, at2_ref[...], preferred_element_type=jnp.float32)
    z14 = jnp.zeros((h2.shape[0], 14), jnp.float32)
    z15 = jnp.zeros((h2.shape[0], 15), jnp.float32)
    h2a_ref[...] = jnp.concatenate([h2, a2, z14], axis=1)
    ad2_ref[...] = jnp.concatenate([a2[:, 1:2], z15], axis=1)


def _tc3_body(q_ref, b2_ref, o_ref):
    q0 = q_ref[0]
    q1 = q_ref[1]
    num = q0[:, :16] + q1[:, :16]
    den = q0[:, 16:17] + q1[:, 16:17]
    o = num / (den + EPS) + b2_ref[...]
    m = jnp.max(o, axis=1, keepdims=True)
    z = o - m
    lse = jnp.log(jnp.sum(jnp.exp(z), axis=1, keepdims=True))
    o_ref[...] = z - lse


_MESH = plsc.VectorSubcoreMesh(core_axis_name="c", subcore_axis_name="s")

_GDN = lax.GatherDimensionNumbers(
    offset_dims=(), collapsed_slice_dims=(0,), start_index_map=(0,))


def _vgat(v, idx):
    return lax.gather(v, idx[:, None], _GDN, (1,),
                      mode=lax.GatherScatterMode.PROMISE_IN_BOUNDS)


def _build_sc(wf, wm, nrnd, split_edges, emit):
    grp = nrnd // 2

    @functools.partial(
        pl.kernel,
        mesh=_MESH,
        compiler_params=pltpu.CompilerParams(use_tc_tiling_on_sc=False),
        out_type=jax.ShapeDtypeStruct((2, NR, wm), jnp.float32),
        scratch_types=[
            pltpu.VMEM((2, SUB, SUBLEN), jnp.int32),
            pltpu.VMEM((2, SUB, SUBLEN), jnp.int32),
            pltpu.VMEM((2, K, wf), jnp.float32),
            pltpu.VMEM((2, K, WAD), jnp.float32),
            pltpu.VMEM((K, wm), jnp.float32),
            pltpu.VMEM_SHARED((NR, wm), jnp.float32),
            pltpu.SemaphoreType.DMA,
            pltpu.SemaphoreType.DMA,
            pltpu.SemaphoreType.DMA,
            pltpu.SemaphoreType.DMA,
            pltpu.SemaphoreType.DMA,
            pltpu.SemaphoreType.DMA,
            pltpu.SemaphoreType.DMA,
        ],
    )
    def sc(src_hbm, dst_hbm, feat_hbm, ad_hbm, out_hbm,
           gis, gid, fb, ab, msg, acc,
           h0, h1, a0, a1, ig0, ig1, ss):
        c = lax.axis_index("c")
        s = lax.axis_index("s")
        sem_h = (h0, h1)
        sem_a = (a0, a1)
        sem_ig = (ig0, ig1)
        if split_edges:
            fsrc = feat_hbm
            asrc = ad_hbm
            ebase = (c * 16 + s) * (nrnd * SUB)
        else:
            fsrc = feat_hbm.at[c]
            asrc = ad_hbm.at[c]
            ebase = s * (nrnd * SUB)

        z = jnp.zeros((16,), jnp.float32)

        def zb(i, _):
            for q in range(wm // 16):
                msg[i, pl.ds(q * 16, 16)] = z
            return 0

        lax.fori_loop(0, K, zb, 0)
        r0 = s * ROWS_PT
        pltpu.sync_copy(msg, acc.at[pl.ds(r0, K)])
        pltpu.sync_copy(msg.at[pl.ds(0, ROWS_PT - K)],
                        acc.at[pl.ds(r0 + K, ROWS_PT - K)])
        plsc.subcore_barrier()

        def fire_gidx(r, sl):
            row = ebase + r * SUB
            pltpu.async_copy(src_hbm.at[pl.ds(row, SUB)], gis.at[sl],
                             sem_ig[sl])
            pltpu.async_copy(dst_hbm.at[pl.ds(row, SUB)], gid.at[sl],
                             sem_ig[sl])

        def drain_gidx(sl):
            pltpu.make_async_copy(src_hbm.at[pl.ds(0, SUB)], gis.at[sl],
                                  sem_ig[sl]).wait()
            pltpu.make_async_copy(dst_hbm.at[pl.ds(0, SUB)], gid.at[sl],
                                  sem_ig[sl]).wait()

        def fire_g(sl):
            for j in range(SUB):
                pltpu.async_copy(fsrc.at[gis.at[sl, j]],
                                 fb.at[sl].at[pl.ds(j * SUBLEN, SUBLEN)],
                                 sem_h[sl])
                pltpu.async_copy(asrc.at[gid.at[sl, j]],
                                 ab.at[sl].at[pl.ds(j * SUBLEN, SUBLEN)],
                                 sem_a[sl])

        def drain_g(sl):
            for j in range(SUB):
                pltpu.make_async_copy(
                    fsrc.at[pl.ds(0, SUBLEN)],
                    fb.at[sl].at[pl.ds(j * SUBLEN, SUBLEN)],
                    sem_h[sl]).wait()
                pltpu.make_async_copy(
                    asrc.at[pl.ds(0, SUBLEN)],
                    ab.at[sl].at[pl.ds(j * SUBLEN, SUBLEN)],
                    sem_a[sl]).wait()

        def scatter(sl):
            dl = []
            for j in range(SUB):
                dl.append(pltpu.async_copy(
                    msg.at[pl.ds(j * SUBLEN, SUBLEN)],
                    acc.at[gid.at[sl, j]], ss, add=True))
            for d in dl:
                d.wait()

        def compute(sl):
            fbx = fb.at[sl]
            abx = ab.at[sl]

            def eb(ii, _):
                for u in range(4):
                    emit(fbx, abx, ii * 4 + u, msg)
                return 0

            lax.fori_loop(0, K // 4, eb, 0)

        def round_steps(r, sl, has_next, has_next2):
            drain_g(sl)
            if has_next is not None:
                def fire_next():
                    drain_gidx(1 - sl)
                    fire_g(1 - sl)
                if has_next is True:
                    fire_next()
                else:
                    pl.when(has_next)(fire_next)
            compute(sl)
            scatter(sl)
            if has_next2 is not None:
                def fire_idx2():
                    fire_gidx(r + 2, sl)
                if has_next2 is True:
                    fire_idx2()
                else:
                    pl.when(has_next2)(fire_idx2)

        fire_gidx(0, 0)
        drain_gidx(0)
        fire_g(0)
        fire_gidx(1, 1)

        def super_body(g, _):
            nl = g < grp - 1
            round_steps(2 * g, 0, True, nl)
            round_steps(2 * g + 1, 1, nl, nl)
            return 0

        lax.fori_loop(0, grp, super_body, 0)
        plsc.subcore_barrier()

        r0 = s * ROWS_PT
        pltpu.sync_copy(acc.at[pl.ds(r0, K)], out_hbm.at[c].at[pl.ds(r0, K)])
        pltpu.sync_copy(acc.at[pl.ds(r0 + K, ROWS_PT - K)],
                        out_hbm.at[c].at[pl.ds(r0 + K, ROWS_PT - K)])

    return sc


def _emit1(fbx, abx, i, msx):
    lane = lax.iota(jnp.int32, 16)
    hsel = lax.shift_right_logical(lane, 3)
    sa = fbx[i, pl.ds(32, 16)]
    da = abx[i, pl.ds(0, 16)]
    e = jnp.minimum(sa + da, 60.0)
    e = jnp.where(e >= 0.0, e, 0.2 * e)
    ex = jnp.exp(e)
    m0 = fbx[i, pl.ds(0, 16)] * _vgat(ex, hsel)
    m1 = fbx[i, pl.ds(16, 16)] * _vgat(ex, hsel + 2)
    msx[i, pl.ds(0, 16)] = m0
    msx[i, pl.ds(16, 16)] = m1
    msx[i, pl.ds(32, 16)] = ex


def _emit2(fbx, abx, i, msx):
    zsel = jnp.zeros((16,), jnp.int32)
    sa = fbx[i, pl.ds(16, 16)]
    da = abx[i, pl.ds(0, 16)]
    e = jnp.minimum(sa + da, 60.0)
    e = jnp.where(e >= 0.0, e, 0.2 * e)
    ex = jnp.exp(e)
    msx[i, pl.ds(0, 16)] = fbx[i, pl.ds(0, 16)] * _vgat(ex, zsel)
    msx[i, pl.ds(16, 16)] = ex


_sc1 = _build_sc(W1A, W1A, EPAD // 16 // K, False, _emit1)
_sc2 = _build_sc(W2A, W2A, EPAD // 32 // K, True, _emit2)


def kernel(x, edge_index, W1, att_src1, att_dst1, b1, W2, att_src2, att_dst2, b2):
    f32 = jnp.float32
    xp = jnp.pad(x.astype(f32), ((0, NR - N), (0, 0)))
    pad = jnp.full((EPAD - E,), N, jnp.int32)
    src2d = jnp.concatenate([edge_index[0], pad]).reshape(IDXROWS, SUBLEN)
    dst2d = jnp.concatenate([edge_index[1], pad]).reshape(IDXROWS, SUBLEN)

    r64 = jnp.arange(64)
    a1 = jnp.zeros((64, 16), f32)
    a1 = a1.at[r64, r64 // 8].set(att_src1.reshape(64))
    a1 = a1.at[r64, 8 + r64 // 8].set(att_dst1.reshape(64))
    at2 = jnp.concatenate([att_src2, att_dst2], axis=0).T

    h1s, ads = pl.pallas_call(
        _tc1_body,
        grid=(_GRID,),
        in_specs=[
            pl.BlockSpec((_BLK, D), lambda i: (i, 0)),
            pl.BlockSpec((D, 64), lambda i: (0, 0)),
            pl.BlockSpec((64, 16), lambda i: (0, 0)),
        ],
        out_specs=[
            pl.BlockSpec((2, _BLK, W1A), lambda i: (0, i, 0)),
            pl.BlockSpec((2, _BLK, WAD), lambda i: (0, i, 0)),
        ],
        out_shape=[
            jax.ShapeDtypeStruct((2, NR, W1A), f32),
            jax.ShapeDtypeStruct((2, NR, WAD), f32),
        ],
    )(xp, W1, a1)

    p = _sc1(src2d, dst2d, h1s, ads)

    h2a, ad2 = pl.pallas_call(
        _tc2_body,
        grid=(_GRID,),
        in_specs=[
            pl.BlockSpec((2, _BLK, W1A), lambda i: (0, i, 0)),
            pl.BlockSpec((1, 64), lambda i: (0, 0)),
            pl.BlockSpec((64, C), lambda i: (0, 0)),
            pl.BlockSpec((C, 2), lambda i: (0, 0)),
        ],
        out_specs=[
            pl.BlockSpec((_BLK, W2A), lambda i: (i, 0)),
            pl.BlockSpec((_BLK, WAD), lambda i: (i, 0)),
        ],
        out_shape=[
            jax.ShapeDtypeStruct((NR, W2A), f32),
            jax.ShapeDtypeStruct((NR, WAD), f32),
        ],
    )(p, b1.reshape(1, 64), W2, at2)

    q = _sc2(src2d, dst2d, h2a, ad2)

    out = pl.pallas_call(
        _tc3_body,
        grid=(_GRID,),
        in_specs=[
            pl.BlockSpec((2, _BLK, W2A), lambda i: (0, i, 0)),
            pl.BlockSpec((1, C), lambda i: (0, 0)),
        ],
        out_specs=pl.BlockSpec((_BLK, C), lambda i: (i, 0)),
        out_shape=jax.ShapeDtypeStruct((NR, C), f32),
    )(q, b2.reshape(1, C))

    return out[:N]


# speedup vs baseline: 97.9724x; 1.4680x over previous
import functools

import jax
import jax.numpy as jnp
from jax import lax
from jax.experimental import pallas as pl
from jax.experimental.pallas import tpu as pltpu
from jax.experimental.pallas import tpu_sc as plsc

N = 10000
E = 320000
D = 128
HID = 8
HEADS = 8
C = 16

NR = 10240
EPAD = 327680
SUBLEN = 128
SUB = 4
K = SUB * SUBLEN
IDXROWS = EPAD // SUBLEN
ROWS_PT = NR // 16

W1A = 48
W1M = 48
WAD = 16
W2A = 32
W2M = 32
EPS = 1e-16

_BLK = 512
_GRID = NR // _BLK


def _tc1_body(x_ref, w1_ref, a1_ref, h1s_ref, ads_ref):
    h = jnp.dot(x_ref[...], w1_ref[...], preferred_element_type=jnp.float32)
    s = jnp.dot(h, a1_ref[...], preferred_element_type=jnp.float32)
    z12 = jnp.zeros((h.shape[0], 12), jnp.float32)
    h1s_ref[0] = jnp.concatenate([h[:, :32], s[:, 0:4], z12], axis=1)
    h1s_ref[1] = jnp.concatenate([h[:, 32:64], s[:, 4:8], z12], axis=1)
    ads_ref[0] = jnp.concatenate([s[:, 8:12], z12], axis=1)
    ads_ref[1] = jnp.concatenate([s[:, 12:16], z12], axis=1)


def _tc2_body(p_ref, b1_ref, w2_ref, at2_ref, h2a_ref, ad2_ref):
    p0 = p_ref[0]
    p1 = p_ref[1]
    num = jnp.concatenate([p0[:, :32], p1[:, :32]], axis=1)
    den = jnp.concatenate([p0[:, 32:36], p1[:, 32:36]], axis=1)
    hh = lax.broadcasted_iota(jnp.int32, (8, 64), 0)
    cc = lax.broadcasted_iota(jnp.int32, (8, 64), 1) // 8
    rep = (hh == cc).astype(jnp.float32)
    den64 = jnp.dot(den, rep, preferred_element_type=jnp.float32)
    out1 = num / (den64 + EPS) + b1_ref[...]
    x2 = jnp.where(out1 > 0, out1, jnp.exp(jnp.minimum(out1, 0.0)) - 1.0)
    h2 = jnp.dot(x2, w2_ref[...], preferred_element_type=jnp.float32)
    a2 = jnp.dot(h2, at2_ref[...], preferred_element_type=jnp.float32)
    z14 = jnp.zeros((h2.shape[0], 14), jnp.float32)
    z15 = jnp.zeros((h2.shape[0], 15), jnp.float32)
    h2a_ref[...] = jnp.concatenate([h2, a2, z14], axis=1)
    ad2_ref[...] = jnp.concatenate([a2[:, 1:2], z15], axis=1)


def _tc3_body(q_ref, b2_ref, o_ref):
    q0 = q_ref[0]
    q1 = q_ref[1]
    num = q0[:, :16] + q1[:, :16]
    den = q0[:, 16:17] + q1[:, 16:17]
    o = num / (den + EPS) + b2_ref[...]
    m = jnp.max(o, axis=1, keepdims=True)
    z = o - m
    lse = jnp.log(jnp.sum(jnp.exp(z), axis=1, keepdims=True))
    o_ref[...] = z - lse


_MESH = plsc.VectorSubcoreMesh(core_axis_name="c", subcore_axis_name="s")

_GDN = lax.GatherDimensionNumbers(
    offset_dims=(), collapsed_slice_dims=(0,), start_index_map=(0,))


def _vgat(v, idx):
    return lax.gather(v, idx[:, None], _GDN, (1,),
                      mode=lax.GatherScatterMode.PROMISE_IN_BOUNDS)


def _build_sc(wf, wm, sub, nrnd, split_edges, emit):
    grp = nrnd // 2
    k = sub * SUBLEN

    @functools.partial(
        pl.kernel,
        mesh=_MESH,
        compiler_params=pltpu.CompilerParams(use_tc_tiling_on_sc=False),
        out_type=jax.ShapeDtypeStruct((2, NR, wm), jnp.float32),
        scratch_types=[
            pltpu.VMEM((2, sub, SUBLEN), jnp.int32),
            pltpu.VMEM((2, sub, SUBLEN), jnp.int32),
            pltpu.VMEM((2, sub, SUBLEN), jnp.int32),
            pltpu.VMEM((2, k, wf), jnp.float32),
            pltpu.VMEM((2, k, WAD), jnp.float32),
            pltpu.VMEM((2, k, wm), jnp.float32),
            pltpu.VMEM_SHARED((NR, wm), jnp.float32),
            pltpu.SemaphoreType.DMA,
            pltpu.SemaphoreType.DMA,
            pltpu.SemaphoreType.DMA,
            pltpu.SemaphoreType.DMA,
            pltpu.SemaphoreType.DMA,
            pltpu.SemaphoreType.DMA,
            pltpu.SemaphoreType.DMA,
            pltpu.SemaphoreType.DMA,
            pltpu.SemaphoreType.DMA,
            pltpu.SemaphoreType.DMA,
        ],
    )
    def sc(src_hbm, dst_hbm, feat_hbm, ad_hbm, out_hbm,
           gis, gid, sid, fb, ab, msg, acc,
           h0, h1, a0, a1, ig0, ig1, is0, is1, s0, s1):
        c = lax.axis_index("c")
        s = lax.axis_index("s")
        sem_h = (h0, h1)
        sem_a = (a0, a1)
        sem_ig = (ig0, ig1)
        sem_is = (is0, is1)
        sem_s = (s0, s1)
        if split_edges:
            fsrc = feat_hbm
            asrc = ad_hbm
            ebase = (c * 16 + s) * (nrnd * sub)
        else:
            fsrc = feat_hbm.at[c]
            asrc = ad_hbm.at[c]
            ebase = s * (nrnd * sub)

        z = jnp.zeros((16,), jnp.float32)

        def zb(i, _):
            for q in range(wm // 16):
                msg[0, i, pl.ds(q * 16, 16)] = z
            return 0

        lax.fori_loop(0, k, zb, 0)
        r0 = s * ROWS_PT
        off = 0
        while off < ROWS_PT:
            n = min(k, ROWS_PT - off)
            pltpu.sync_copy(msg.at[0].at[pl.ds(0, n)],
                            acc.at[pl.ds(r0 + off, n)])
            off += n
        plsc.subcore_barrier()

        def fire_gidx(r, sl):
            row = ebase + r * sub
            pltpu.async_copy(src_hbm.at[pl.ds(row, sub)], gis.at[sl],
                             sem_ig[sl])
            pltpu.async_copy(dst_hbm.at[pl.ds(row, sub)], gid.at[sl],
                             sem_ig[sl])

        def drain_gidx(sl):
            pltpu.make_async_copy(src_hbm.at[pl.ds(0, sub)], gis.at[sl],
                                  sem_ig[sl]).wait()
            pltpu.make_async_copy(dst_hbm.at[pl.ds(0, sub)], gid.at[sl],
                                  sem_ig[sl]).wait()

        def fire_sidx(r, sl):
            row = ebase + r * sub
            pltpu.async_copy(dst_hbm.at[pl.ds(row, sub)], sid.at[sl],
                             sem_is[sl])

        def drain_sidx(sl):
            pltpu.make_async_copy(dst_hbm.at[pl.ds(0, sub)], sid.at[sl],
                                  sem_is[sl]).wait()

        def fire_g(sl):
            for j in range(sub):
                pltpu.async_copy(fsrc.at[gis.at[sl, j]],
                                 fb.at[sl].at[pl.ds(j * SUBLEN, SUBLEN)],
                                 sem_h[sl])
                pltpu.async_copy(asrc.at[gid.at[sl, j]],
                                 ab.at[sl].at[pl.ds(j * SUBLEN, SUBLEN)],
                                 sem_a[sl])

        def drain_g(sl):
            for j in range(sub):
                pltpu.make_async_copy(
                    fsrc.at[pl.ds(0, SUBLEN)],
                    fb.at[sl].at[pl.ds(j * SUBLEN, SUBLEN)],
                    sem_h[sl]).wait()
                pltpu.make_async_copy(
                    asrc.at[pl.ds(0, SUBLEN)],
                    ab.at[sl].at[pl.ds(j * SUBLEN, SUBLEN)],
                    sem_a[sl]).wait()

        def fire_sc(sl):
            for j in range(sub):
                pltpu.async_copy(msg.at[sl].at[pl.ds(j * SUBLEN, SUBLEN)],
                                 acc.at[sid.at[sl, j]], sem_s[sl], add=True)

        def drain_sc(sl):
            for j in range(sub):
                pltpu.make_async_copy(
                    msg.at[sl].at[pl.ds(j * SUBLEN, SUBLEN)],
                    acc.at[pl.ds(0, SUBLEN)], sem_s[sl]).wait()

        def compute(sl):
            fbx = fb.at[sl]
            abx = ab.at[sl]
            msx = msg.at[sl]

            @plsc.parallel_loop(0, k, unroll=16)
            def _(i):
                emit(fbx, abx, i, msx)

        def _guard(cond, fn):
            if cond is True:
                fn()
            else:
                pl.when(cond)(fn)

        def round_steps(x, sl, first, gn, g2, sn):
            drain_g(sl)

            def _fg():
                drain_gidx(1 - sl)
                fire_g(1 - sl)
            _guard(gn, _fg)
            _guard(g2, lambda: fire_gidx(x + 2, sl))
            compute(sl)
            if first is None:
                drain_sc(1 - sl)
            else:
                pl.when(first)(lambda: drain_sc(1 - sl))
            _guard(sn, lambda: fire_sidx(x + 1, 1 - sl))
            drain_sidx(sl)
            fire_sc(sl)

        fire_gidx(0, 0)
        fire_sidx(0, 0)
        drain_gidx(0)
        fire_g(0)
        fire_gidx(1, 1)

        def super_body(g, _):
            nl = g < grp - 1
            round_steps(2 * g, 0, g > 0, True, nl, True)
            round_steps(2 * g + 1, 1, None, nl, nl, nl)
            return 0

        lax.fori_loop(0, grp, super_body, 0)
        drain_sc(1)
        plsc.subcore_barrier()

        r0 = s * ROWS_PT
        pltpu.sync_copy(acc.at[pl.ds(r0, K)], out_hbm.at[c].at[pl.ds(r0, K)])
        pltpu.sync_copy(acc.at[pl.ds(r0 + K, ROWS_PT - K)],
                        out_hbm.at[c].at[pl.ds(r0 + K, ROWS_PT - K)])

    return sc


def _emit1(fbx, abx, i, msx):
    lane = lax.iota(jnp.int32, 16)
    hsel = lax.shift_right_logical(lane, 3)
    sa = fbx[i, pl.ds(32, 16)]
    da = abx[i, pl.ds(0, 16)]
    e = jnp.minimum(sa + da, 60.0)
    e = jnp.where(e >= 0.0, e, 0.2 * e)
    ex = jnp.exp(e)
    msx[i, pl.ds(0, 16)] = fbx[i, pl.ds(0, 16)] * _vgat(ex, hsel)
    msx[i, pl.ds(16, 16)] = fbx[i, pl.ds(16, 16)] * _vgat(ex, hsel + 2)
    msx[i, pl.ds(32, 16)] = ex


def _emit2(fbx, abx, i, msx):
    zsel = jnp.zeros((16,), jnp.int32)
    sa = fbx[i, pl.ds(16, 16)]
    da = abx[i, pl.ds(0, 16)]
    e = jnp.minimum(sa + da, 60.0)
    e = jnp.where(e >= 0.0, e, 0.2 * e)
    ex = jnp.exp(e)
    msx[i, pl.ds(0, 16)] = fbx[i, pl.ds(0, 16)] * _vgat(ex, zsel)
    msx[i, pl.ds(16, 16)] = ex


_sc1 = _build_sc(W1A, W1M, 2, EPAD // 16 // 256, False, _emit1)
_sc2 = _build_sc(W2A, W2M, 4, EPAD // 32 // K, True, _emit2)


def kernel(x, edge_index, W1, att_src1, att_dst1, b1, W2, att_src2, att_dst2, b2):
    f32 = jnp.float32
    xp = jnp.pad(x.astype(f32), ((0, NR - N), (0, 0)))
    pad = jnp.full((EPAD - E,), N, jnp.int32)
    src2d = jnp.concatenate([edge_index[0], pad]).reshape(IDXROWS, SUBLEN)
    dst2d = jnp.concatenate([edge_index[1], pad]).reshape(IDXROWS, SUBLEN)

    r64 = jnp.arange(64)
    a1 = jnp.zeros((64, 16), f32)
    a1 = a1.at[r64, r64 // 8].set(att_src1.reshape(64))
    a1 = a1.at[r64, 8 + r64 // 8].set(att_dst1.reshape(64))
    at2 = jnp.concatenate([att_src2, att_dst2], axis=0).T

    h1s, ads = pl.pallas_call(
        _tc1_body,
        grid=(_GRID,),
        in_specs=[
            pl.BlockSpec((_BLK, D), lambda i: (i, 0)),
            pl.BlockSpec((D, 64), lambda i: (0, 0)),
            pl.BlockSpec((64, 16), lambda i: (0, 0)),
        ],
        out_specs=[
            pl.BlockSpec((2, _BLK, W1A), lambda i: (0, i, 0)),
            pl.BlockSpec((2, _BLK, WAD), lambda i: (0, i, 0)),
        ],
        out_shape=[
            jax.ShapeDtypeStruct((2, NR, W1A), f32),
            jax.ShapeDtypeStruct((2, NR, WAD), f32),
        ],
    )(xp, W1, a1)

    p = _sc1(src2d, dst2d, h1s, ads)

    h2a, ad2 = pl.pallas_call(
        _tc2_body,
        grid=(_GRID,),
        in_specs=[
            pl.BlockSpec((2, _BLK, W1M), lambda i: (0, i, 0)),
            pl.BlockSpec((1, 64), lambda i: (0, 0)),
            pl.BlockSpec((64, C), lambda i: (0, 0)),
            pl.BlockSpec((C, 2), lambda i: (0, 0)),
        ],
        out_specs=[
            pl.BlockSpec((_BLK, W2A), lambda i: (i, 0)),
            pl.BlockSpec((_BLK, WAD), lambda i: (i, 0)),
        ],
        out_shape=[
            jax.ShapeDtypeStruct((NR, W2A), f32),
            jax.ShapeDtypeStruct((NR, WAD), f32),
        ],
    )(p, b1.reshape(1, 64), W2, at2)

    q = _sc2(src2d, dst2d, h2a, ad2)

    out = pl.pallas_call(
        _tc3_body,
        grid=(_GRID,),
        in_specs=[
            pl.BlockSpec((2, _BLK, W2M), lambda i: (0, i, 0)),
            pl.BlockSpec((1, C), lambda i: (0, 0)),
        ],
        out_specs=pl.BlockSpec((_BLK, C), lambda i: (i, 0)),
        out_shape=jax.ShapeDtypeStruct((NR, C), f32),
    )(q, b2.reshape(1, C))

    return out[:N]
